# Initial kernel scaffold; baseline (speedup 1.0000x reference)
import jax
import jax.numpy as jnp
from jax.experimental import pallas as pl


def kernel(t, betas, alphas, alpha_bars, alpha_bars_prev, sqrt_one_minus_alpha_bars, sigmas):
    raise NotImplementedError("write your pallas kernel here")


# speedup vs baseline: 23.6935x; 23.6935x over previous
import functools

import jax
import jax.numpy as jnp
from jax import lax
from jax.experimental import pallas as pl
from jax.experimental.pallas import tpu as pltpu
from jax.experimental.pallas import tpu_sc as plsc

NUM_TABLES = 6
STEPS = 1000
BATCH = 16384
NC = 2
NS = 16
L = 16
NW = NC * NS
CHUNK = BATCH // NW
GROUPS = CHUNK // L


@functools.partial(
    pl.kernel,
    mesh=plsc.VectorSubcoreMesh(core_axis_name="c", subcore_axis_name="s"),
    out_type=jax.ShapeDtypeStruct((NUM_TABLES, BATCH), jnp.float32),
    compiler_params=pltpu.CompilerParams(needs_layout_passes=False),
    scratch_types=[
        pltpu.VMEM((CHUNK,), jnp.int32),
        pltpu.VMEM((NUM_TABLES * STEPS,), jnp.float32),
        pltpu.VMEM((NUM_TABLES * CHUNK,), jnp.float32),
        pltpu.SemaphoreType.DMA,
    ],
)
def _sched_gather(t_hbm, tb0, tb1, tb2, tb3, tb4, tb5, out_hbm,
                  idx_v, tab_v, out_v, sem):
    wid = lax.axis_index("s") * NC + lax.axis_index("c")
    base = pl.multiple_of(wid * CHUNK, CHUNK)
    tabs = (tb0, tb1, tb2, tb3, tb4, tb5)

    loads = [pltpu.async_copy(t_hbm.at[pl.ds(base, CHUNK)], idx_v, sem)]
    for j in range(NUM_TABLES):
        loads.append(
            pltpu.async_copy(tabs[j], tab_v.at[pl.ds(j * STEPS, STEPS)], sem))
    for c in loads:
        c.wait()

    for g in range(GROUPS):
        idx = idx_v[pl.ds(g * L, L)]
        for j in range(NUM_TABLES):
            vals = plsc.load_gather(tab_v, [idx + (j * STEPS)])
            out_v[pl.ds(j * CHUNK + g * L, L)] = vals

    stores = [
        pltpu.async_copy(out_v.at[pl.ds(j * CHUNK, CHUNK)],
                         out_hbm.at[j, pl.ds(base, CHUNK)], sem)
        for j in range(NUM_TABLES)
    ]
    for c in stores:
        c.wait()


def kernel(t, betas, alphas, alpha_bars, alpha_bars_prev,
           sqrt_one_minus_alpha_bars, sigmas):
    return _sched_gather(t, betas, alphas, alpha_bars, alpha_bars_prev,
                         sqrt_one_minus_alpha_bars, sigmas)


# speedup vs baseline: 23.9705x; 1.0117x over previous
import functools

import jax
import jax.numpy as jnp
from jax import lax
from jax.experimental import pallas as pl
from jax.experimental.pallas import tpu as pltpu
from jax.experimental.pallas import tpu_sc as plsc

NUM_TABLES = 6
STEPS = 1000
BATCH = 16384
NC = 2
NS = 16
L = 16
NW = NC * NS
CHUNK = BATCH // NW
GROUPS = CHUNK // L


@functools.partial(
    pl.kernel,
    mesh=plsc.VectorSubcoreMesh(core_axis_name="c", subcore_axis_name="s"),
    out_type=jax.ShapeDtypeStruct((NUM_TABLES, BATCH), jnp.float32),
    compiler_params=pltpu.CompilerParams(needs_layout_passes=False),
    scratch_types=[
        pltpu.VMEM((CHUNK,), jnp.int32),
        *[pltpu.VMEM((STEPS,), jnp.float32) for _ in range(NUM_TABLES)],
        pltpu.VMEM((NUM_TABLES, CHUNK), jnp.float32),
        pltpu.SemaphoreType.DMA,
    ],
)
def _sched_gather(t_hbm, tb0, tb1, tb2, tb3, tb4, tb5, out_hbm,
                  idx_v, tv0, tv1, tv2, tv3, tv4, tv5, out_v, sem):
    wid = lax.axis_index("s") * NC + lax.axis_index("c")
    base = pl.multiple_of(wid * CHUNK, CHUNK)
    tabs = (tb0, tb1, tb2, tb3, tb4, tb5)
    tvs = (tv0, tv1, tv2, tv3, tv4, tv5)

    loads = [pltpu.async_copy(t_hbm.at[pl.ds(base, CHUNK)], idx_v, sem)]
    for j in range(NUM_TABLES):
        loads.append(pltpu.async_copy(tabs[j], tvs[j], sem))
    for c in loads:
        c.wait()

    for g in range(GROUPS):
        idx = idx_v[pl.ds(g * L, L)]
        for j in range(NUM_TABLES):
            out_v[j, pl.ds(g * L, L)] = plsc.load_gather(tvs[j], [idx])

    pltpu.async_copy(out_v, out_hbm.at[:, pl.ds(base, CHUNK)], sem).wait()


def kernel(t, betas, alphas, alpha_bars, alpha_bars_prev,
           sqrt_one_minus_alpha_bars, sigmas):
    return _sched_gather(t, betas, alphas, alpha_bars, alpha_bars_prev,
                         sqrt_one_minus_alpha_bars, sigmas)


# speedup vs baseline: 24.8249x; 1.0356x over previous
import functools

import jax
import jax.numpy as jnp
from jax import lax
from jax.experimental import pallas as pl
from jax.experimental.pallas import tpu as pltpu
from jax.experimental.pallas import tpu_sc as plsc

NUM_TABLES = 6
STEPS = 1000
BATCH = 16384
NC = 2
NS = 16
L = 16
NW = NC * NS
CHUNK = BATCH // NW
GROUPS = CHUNK // L


@functools.partial(
    pl.kernel,
    mesh=plsc.VectorSubcoreMesh(core_axis_name="c", subcore_axis_name="s"),
    out_type=jax.ShapeDtypeStruct((NUM_TABLES, BATCH), jnp.float32),
    compiler_params=pltpu.CompilerParams(needs_layout_passes=False),
    scratch_types=[
        pltpu.VMEM((CHUNK,), jnp.int32),
        *[pltpu.VMEM((STEPS,), jnp.float32) for _ in range(NUM_TABLES)],
        pltpu.VMEM((NUM_TABLES, CHUNK), jnp.float32),
        pltpu.SemaphoreType.DMA,
    ],
)
def _sched_gather(t_hbm, tb0, tb1, tb2, tb3, tb4, tb5, out_hbm,
                  idx_v, tv0, tv1, tv2, tv3, tv4, tv5, out_v, sem):
    wid = lax.axis_index("s") * NC + lax.axis_index("c")
    base = pl.multiple_of(wid * CHUNK, CHUNK)
    tabs = (tb0, tb1, tb2, tb3, tb4, tb5)
    tvs = (tv0, tv1, tv2, tv3, tv4, tv5)

    loads = [pltpu.async_copy(t_hbm.at[pl.ds(base, CHUNK)], idx_v, sem)]
    for j in range(NUM_TABLES):
        loads.append(pltpu.async_copy(tabs[j], tvs[j], sem))
    for c in loads:
        c.wait()

    def body(g, carry):
        off = pl.multiple_of(g * L, L)
        idx = idx_v[pl.ds(off, L)]
        for j in range(NUM_TABLES):
            out_v[j, pl.ds(off, L)] = plsc.load_gather(tvs[j], [idx])
        return carry

    lax.fori_loop(0, GROUPS, body, 0, unroll=2)

    pltpu.async_copy(out_v, out_hbm.at[:, pl.ds(base, CHUNK)], sem).wait()


def kernel(t, betas, alphas, alpha_bars, alpha_bars_prev,
           sqrt_one_minus_alpha_bars, sigmas):
    return _sched_gather(t, betas, alphas, alpha_bars, alpha_bars_prev,
                         sqrt_one_minus_alpha_bars, sigmas)


# speedup vs baseline: 26.6092x; 1.0719x over previous
import functools

import jax
import jax.numpy as jnp
from jax import lax
from jax.experimental import pallas as pl
from jax.experimental.pallas import tpu as pltpu
from jax.experimental.pallas import tpu_sc as plsc

NUM_TABLES = 6
STEPS = 1000
BATCH = 16384
NC = 1
NS = 16
L = 16
NW = NC * NS
CHUNK = BATCH // NW
GROUPS = CHUNK // L


@functools.partial(
    pl.kernel,
    mesh=plsc.VectorSubcoreMesh(core_axis_name="c", subcore_axis_name="s",
                                num_cores=NC),
    out_type=jax.ShapeDtypeStruct((NUM_TABLES, BATCH), jnp.float32),
    compiler_params=pltpu.CompilerParams(needs_layout_passes=False),
    scratch_types=[
        pltpu.VMEM((CHUNK,), jnp.int32),
        *[pltpu.VMEM((STEPS,), jnp.float32) for _ in range(NUM_TABLES)],
        pltpu.VMEM((NUM_TABLES, CHUNK), jnp.float32),
        pltpu.SemaphoreType.DMA,
    ],
)
def _sched_gather(t_hbm, tb0, tb1, tb2, tb3, tb4, tb5, out_hbm,
                  idx_v, tv0, tv1, tv2, tv3, tv4, tv5, out_v, sem):
    wid = lax.axis_index("s") * NC + lax.axis_index("c")
    base = pl.multiple_of(wid * CHUNK, CHUNK)
    tabs = (tb0, tb1, tb2, tb3, tb4, tb5)
    tvs = (tv0, tv1, tv2, tv3, tv4, tv5)

    loads = [pltpu.async_copy(t_hbm.at[pl.ds(base, CHUNK)], idx_v, sem)]
    for j in range(NUM_TABLES):
        loads.append(pltpu.async_copy(tabs[j], tvs[j], sem))
    for c in loads:
        c.wait()

    def body(g, carry):
        off = pl.multiple_of(g * L, L)
        idx = idx_v[pl.ds(off, L)]
        for j in range(NUM_TABLES):
            out_v[j, pl.ds(off, L)] = plsc.load_gather(tvs[j], [idx])
        return carry

    lax.fori_loop(0, GROUPS, body, 0, unroll=2)

    pltpu.async_copy(out_v, out_hbm.at[:, pl.ds(base, CHUNK)], sem).wait()


def kernel(t, betas, alphas, alpha_bars, alpha_bars_prev,
           sqrt_one_minus_alpha_bars, sigmas):
    return _sched_gather(t, betas, alphas, alpha_bars, alpha_bars_prev,
                         sqrt_one_minus_alpha_bars, sigmas)
